# baseline (speedup 1.0000x reference)
import functools
import math

import jax
import jax.numpy as jnp
from jax import lax
from jax.experimental import pallas as pl
from jax.experimental.pallas import tpu as pltpu

_NUM_BINS = 512
_BIN_RES = 0.01
_T0 = 0.0
_HALF = _BIN_RES / 2
_NP = 512
_CHUNK = 128
_NCHUNK = _NP // _CHUNK
_LOG2E = 1.4426950408889634
_SQ_HALF_PI = math.sqrt(0.5 / math.pi)


def _hist_kernel(scan_ref, fields_ref, out_ref, acc_ref, *, steps):
    j = pl.program_id(0)

    @pl.when(j == 0)
    def _():
        acc_ref[...] = jnp.zeros_like(acc_ref)

    r_bc = (lax.broadcasted_iota(jnp.int32, (_NUM_BINS, _CHUNK), 0) + 1
            ).astype(jnp.float32) * _HALF + (_T0 / 2)

    sx = scan_ref[0]
    sy = scan_ref[1]
    sz = scan_ref[2]

    acc = acc_ref[...]
    for c in range(_NCHUNK):
        f = fields_ref[:, c * _CHUNK:(c + 1) * _CHUNK]
        dx = f[0:1, :] - sx
        dy = f[1:2, :] - sy
        dz = f[2:3, :] - sz
        r0 = jnp.sqrt(dx * dx + dy * dy + dz * dz)
        colour = f[3:4, :]
        coefv = f[4:5, :]
        opac = f[5:6, :]
        scalev = f[6:7, :]
        sigma = jnp.maximum(jnp.exp(scalev), _HALF)
        isig = 1.0 / sigma
        coeff = 1.0 / (1.0 + jnp.exp(-coefv))
        amp = (opac * opac) * (colour * colour) * _HALF
        a_row = amp * coeff * _SQ_HALF_PI * isig
        b_row = amp * (1.0 - coeff) * (isig * isig)
        c2 = (-0.5 * _LOG2E) * (isig * isig)

        u = r_bc - r0
        q = u * u
        e = jnp.exp2(q * c2)
        w = a_row + b_row * u
        acc = acc + jnp.maximum(e * w, 0.0)
    acc_ref[...] = acc

    @pl.when(j == steps - 1)
    def _():
        r_col = (lax.broadcasted_iota(jnp.int32, (_NUM_BINS, 1), 0) + 1
                 ).astype(jnp.float32) * _HALF + (_T0 / 2)
        hist = jnp.sum(acc_ref[...], axis=1, keepdims=True)
        out_ref[:, :] = hist / (r_col * r_col)


def _run_shard(scan_point, fields):
    steps = fields.shape[1] // _NP
    out = pl.pallas_call(
        functools.partial(_hist_kernel, steps=steps),
        grid=(steps,),
        in_specs=[
            pl.BlockSpec(memory_space=pltpu.SMEM),
            pl.BlockSpec((8, _NP), lambda j: (0, j)),
        ],
        out_specs=pl.BlockSpec((_NUM_BINS, 1), lambda j: (0, 0)),
        out_shape=jax.ShapeDtypeStruct((_NUM_BINS, 1), jnp.float32),
        scratch_shapes=[pltpu.VMEM((_NUM_BINS, _CHUNK), jnp.float32)],
        compiler_params=pltpu.CompilerParams(
            dimension_semantics=("arbitrary",)),
    )(scan_point, fields)
    return out[:, 0]


def kernel(means, scan_point, colours, coefficients, opacities, scales,
           view_id):
    n = means.shape[0]
    opac = jnp.take(opacities, view_id, axis=1)
    fields = jnp.stack([
        means[:, 0], means[:, 1], means[:, 2],
        colours[:, 0], coefficients[:, 0], opac, scales[:, 0],
    ], axis=0)

    devs = jax.devices()
    ncore = 2 if len(devs) >= 2 else 1
    steps = -(-n // (ncore * _NP))
    npad = ncore * _NP * steps
    fields = jnp.pad(fields, ((0, 1), (0, npad - n)))

    if ncore == 1:
        return _run_shard(scan_point, fields)

    mesh = jax.sharding.Mesh(devs[:ncore], ("x",))
    P = jax.sharding.PartitionSpec

    def _shard_fn(scan_l, fields_l):
        return jax.lax.psum(_run_shard(scan_l, fields_l), "x")

    return jax.shard_map(
        _shard_fn, mesh=mesh,
        in_specs=(P(), P(None, "x")),
        out_specs=P(), check_vma=False,
    )(scan_point, fields)


# speedup vs baseline: 1.3997x; 1.3997x over previous
import functools
import math

import jax
import jax.numpy as jnp
from jax import lax
from jax.experimental import pallas as pl
from jax.experimental.pallas import tpu as pltpu

_NUM_BINS = 512
_BIN_RES = 0.01
_T0 = 0.0
_HALF = _BIN_RES / 2
_NP = 2048
_CHUNK = 128
_NCHUNK = _NP // _CHUNK
_LOG2E = 1.4426950408889634
_SQ_HALF_PI = math.sqrt(0.5 / math.pi)


def _hist_kernel(scan_ref, fields_ref, out_ref, acc_ref, *, steps):
    j = pl.program_id(0)

    @pl.when(j == 0)
    def _():
        acc_ref[...] = jnp.zeros_like(acc_ref)

    r_bc = (lax.broadcasted_iota(jnp.int32, (_NUM_BINS, _CHUNK), 0) + 1
            ).astype(jnp.float32) * _HALF + (_T0 / 2)

    sx = scan_ref[0]
    sy = scan_ref[1]
    sz = scan_ref[2]

    acc = acc_ref[...]
    for c in range(_NCHUNK):
        f = fields_ref[:, c * _CHUNK:(c + 1) * _CHUNK]
        dx = f[0:1, :] - sx
        dy = f[1:2, :] - sy
        dz = f[2:3, :] - sz
        r0 = jnp.sqrt(dx * dx + dy * dy + dz * dz)
        colour = f[3:4, :]
        coefv = f[4:5, :]
        opac = f[5:6, :]
        scalev = f[6:7, :]
        sigma = jnp.maximum(jnp.exp(scalev), _HALF)
        isig = 1.0 / sigma
        coeff = 1.0 / (1.0 + jnp.exp(-coefv))
        amp = (opac * opac) * (colour * colour) * _HALF
        a_row = amp * coeff * _SQ_HALF_PI * isig
        b_row = amp * (1.0 - coeff) * (isig * isig)
        c2 = (-0.5 * _LOG2E) * (isig * isig)

        u = r_bc - r0
        q = u * u
        e = jnp.exp2(q * c2)
        w = a_row + b_row * u
        acc = acc + jnp.maximum(e * w, 0.0)
    acc_ref[...] = acc

    @pl.when(j == steps - 1)
    def _():
        r_col = (lax.broadcasted_iota(jnp.int32, (_NUM_BINS, 1), 0) + 1
                 ).astype(jnp.float32) * _HALF + (_T0 / 2)
        hist = jnp.sum(acc_ref[...], axis=1, keepdims=True)
        out_ref[:, :] = hist / (r_col * r_col)


def _run_shard(scan_point, fields):
    steps = fields.shape[1] // _NP
    out = pl.pallas_call(
        functools.partial(_hist_kernel, steps=steps),
        grid=(steps,),
        in_specs=[
            pl.BlockSpec(memory_space=pltpu.SMEM),
            pl.BlockSpec((8, _NP), lambda j: (0, j)),
        ],
        out_specs=pl.BlockSpec((_NUM_BINS, 1), lambda j: (0, 0)),
        out_shape=jax.ShapeDtypeStruct((_NUM_BINS, 1), jnp.float32),
        scratch_shapes=[pltpu.VMEM((_NUM_BINS, _CHUNK), jnp.float32)],
        compiler_params=pltpu.CompilerParams(
            dimension_semantics=("arbitrary",)),
    )(scan_point, fields)
    return out[:, 0]


def kernel(means, scan_point, colours, coefficients, opacities, scales,
           view_id):
    n = means.shape[0]
    opac = jnp.take(opacities, view_id, axis=1)
    fields = jnp.stack([
        means[:, 0], means[:, 1], means[:, 2],
        colours[:, 0], coefficients[:, 0], opac, scales[:, 0],
    ], axis=0)

    devs = jax.devices()
    ncore = 2 if len(devs) >= 2 else 1
    steps = -(-n // (ncore * _NP))
    npad = ncore * _NP * steps
    fields = jnp.pad(fields, ((0, 1), (0, npad - n)))

    if ncore == 1:
        return _run_shard(scan_point, fields)

    mesh = jax.sharding.Mesh(devs[:ncore], ("x",))
    P = jax.sharding.PartitionSpec

    def _shard_fn(scan_l, fields_l):
        return jax.lax.psum(_run_shard(scan_l, fields_l), "x")

    return jax.shard_map(
        _shard_fn, mesh=mesh,
        in_specs=(P(), P(None, "x")),
        out_specs=P(), check_vma=False,
    )(scan_point, fields)


# speedup vs baseline: 3.8525x; 2.7524x over previous
import functools
import math

import jax
import jax.numpy as jnp
from jax import lax
from jax.experimental import pallas as pl
from jax.experimental.pallas import tpu as pltpu

_NUM_BINS = 512
_BIN_RES = 0.01
_T0 = 0.0
_HALF = _BIN_RES / 2
_NP = 2048
_CHUNK = 128
_NCHUNK = _NP // _CHUNK
_LOG2E = 1.4426950408889634
_SQ_HALF_PI = math.sqrt(0.5 / math.pi)


def _hist_kernel(scan_ref, fields_ref, out_ref, acc_ref, *, steps):
    j = pl.program_id(0)

    @pl.when(j == 0)
    def _():
        acc_ref[...] = jnp.zeros_like(acc_ref)

    r_bc = (lax.broadcasted_iota(jnp.int32, (_NUM_BINS, _CHUNK), 0) + 1
            ).astype(jnp.float32) * _HALF + (_T0 / 2)

    sx = scan_ref[0]
    sy = scan_ref[1]
    sz = scan_ref[2]

    acc = acc_ref[...]
    for c in range(_NCHUNK):
        f = fields_ref[:, c * _CHUNK:(c + 1) * _CHUNK]
        dx = f[0:1, :] - sx
        dy = f[1:2, :] - sy
        dz = f[2:3, :] - sz
        r0 = jnp.sqrt(dx * dx + dy * dy + dz * dz)
        colour = f[3:4, :]
        coefv = f[4:5, :]
        opac = f[5:6, :]
        scalev = f[6:7, :]
        sigma = jnp.maximum(jnp.exp(scalev), _HALF)
        isig = 1.0 / sigma
        coeff = 1.0 / (1.0 + jnp.exp(-coefv))
        amp = (opac * opac) * (colour * colour) * _HALF
        a_row = amp * coeff * _SQ_HALF_PI * isig
        b_row = amp * (1.0 - coeff) * (isig * isig)
        c2 = (-0.5 * _LOG2E) * (isig * isig)

        u = r_bc - r0
        q = u * u
        e = jnp.exp2(q * c2)
        w = a_row + b_row * u
        acc = acc + jnp.maximum(e * w, 0.0)
    acc_ref[...] = acc

    @pl.when(j == steps - 1)
    def _():
        r_col = (lax.broadcasted_iota(jnp.int32, (_NUM_BINS, 1), 0) + 1
                 ).astype(jnp.float32) * _HALF + (_T0 / 2)
        hist = jnp.sum(acc_ref[...], axis=1, keepdims=True)
        out_ref[:, :] = hist / (r_col * r_col)


def _run_shard(scan_point, fields):
    steps = fields.shape[1] // _NP
    out = pl.pallas_call(
        functools.partial(_hist_kernel, steps=steps),
        grid=(steps,),
        in_specs=[
            pl.BlockSpec(memory_space=pltpu.SMEM),
            pl.BlockSpec((8, _NP), lambda j: (0, j)),
        ],
        out_specs=pl.BlockSpec((_NUM_BINS, 1), lambda j: (0, 0)),
        out_shape=jax.ShapeDtypeStruct((_NUM_BINS, 1), jnp.float32),
        scratch_shapes=[pltpu.VMEM((_NUM_BINS, _CHUNK), jnp.float32)],
        compiler_params=pltpu.CompilerParams(
            dimension_semantics=("arbitrary",)),
    )(scan_point, fields)
    return out[:, 0]


def kernel(means, scan_point, colours, coefficients, opacities, scales,
           view_id):
    n = means.shape[0]
    opac = jnp.take(opacities, view_id, axis=1)
    fields = jnp.stack([
        means[:, 0], means[:, 1], means[:, 2],
        colours[:, 0], coefficients[:, 0], opac, scales[:, 0],
    ], axis=0)

    devs = jax.devices()
    ncore = 1
    steps = -(-n // (ncore * _NP))
    npad = ncore * _NP * steps
    fields = jnp.pad(fields, ((0, 1), (0, npad - n)))

    if ncore == 1:
        return _run_shard(scan_point, fields)

    mesh = jax.sharding.Mesh(devs[:ncore], ("x",))
    P = jax.sharding.PartitionSpec

    def _shard_fn(scan_l, fields_l):
        return jax.lax.psum(_run_shard(scan_l, fields_l), "x")

    return jax.shard_map(
        _shard_fn, mesh=mesh,
        in_specs=(P(), P(None, "x")),
        out_specs=P(), check_vma=False,
    )(scan_point, fields)


# speedup vs baseline: 3.9377x; 1.0221x over previous
import functools
import math

import jax
import jax.numpy as jnp
from jax import lax
from jax.experimental import pallas as pl
from jax.experimental.pallas import tpu as pltpu

_NUM_BINS = 512
_BIN_RES = 0.01
_T0 = 0.0
_HALF = _BIN_RES / 2
_NP = 4096
_CHUNK = 128
_NCHUNK = _NP // _CHUNK
_LOG2E = 1.4426950408889634
_SQ_HALF_PI = math.sqrt(0.5 / math.pi)


def _hist_kernel(scan_ref, fields_ref, out_ref, acc_ref, *, steps):
    j = pl.program_id(0)

    @pl.when(j == 0)
    def _():
        acc_ref[...] = jnp.zeros_like(acc_ref)

    r_bc = (lax.broadcasted_iota(jnp.int32, (_NUM_BINS, _CHUNK), 0) + 1
            ).astype(jnp.float32) * _HALF + (_T0 / 2)

    sx = scan_ref[0]
    sy = scan_ref[1]
    sz = scan_ref[2]

    acc = acc_ref[...]
    for c in range(_NCHUNK):
        f = fields_ref[:, c * _CHUNK:(c + 1) * _CHUNK]
        dx = f[0:1, :] - sx
        dy = f[1:2, :] - sy
        dz = f[2:3, :] - sz
        r0 = jnp.sqrt(dx * dx + dy * dy + dz * dz)
        colour = f[3:4, :]
        coefv = f[4:5, :]
        opac = f[5:6, :]
        scalev = f[6:7, :]
        sigma = jnp.maximum(jnp.exp(scalev), _HALF)
        isig = 1.0 / sigma
        coeff = 1.0 / (1.0 + jnp.exp(-coefv))
        amp = (opac * opac) * (colour * colour) * _HALF
        a_row = amp * coeff * _SQ_HALF_PI * isig
        b_row = amp * (1.0 - coeff) * (isig * isig)
        c2 = (-0.5 * _LOG2E) * (isig * isig)

        u = r_bc - r0
        q = u * u
        e = jnp.exp2(q * c2)
        w = a_row + b_row * u
        acc = acc + jnp.maximum(e * w, 0.0)
    acc_ref[...] = acc

    @pl.when(j == steps - 1)
    def _():
        r_col = (lax.broadcasted_iota(jnp.int32, (_NUM_BINS, 1), 0) + 1
                 ).astype(jnp.float32) * _HALF + (_T0 / 2)
        hist = jnp.sum(acc_ref[...], axis=1, keepdims=True)
        out_ref[:, :] = hist / (r_col * r_col)


def _run_shard(scan_point, fields):
    steps = fields.shape[1] // _NP
    out = pl.pallas_call(
        functools.partial(_hist_kernel, steps=steps),
        grid=(steps,),
        in_specs=[
            pl.BlockSpec(memory_space=pltpu.SMEM),
            pl.BlockSpec((8, _NP), lambda j: (0, j)),
        ],
        out_specs=pl.BlockSpec((_NUM_BINS, 1), lambda j: (0, 0)),
        out_shape=jax.ShapeDtypeStruct((_NUM_BINS, 1), jnp.float32),
        scratch_shapes=[pltpu.VMEM((_NUM_BINS, _CHUNK), jnp.float32)],
        compiler_params=pltpu.CompilerParams(
            dimension_semantics=("arbitrary",)),
    )(scan_point, fields)
    return out[:, 0]


def kernel(means, scan_point, colours, coefficients, opacities, scales,
           view_id):
    n = means.shape[0]
    opac = jnp.take(opacities, view_id, axis=1)
    fields = jnp.stack([
        means[:, 0], means[:, 1], means[:, 2],
        colours[:, 0], coefficients[:, 0], opac, scales[:, 0],
    ], axis=0)

    devs = jax.devices()
    ncore = 1
    steps = -(-n // (ncore * _NP))
    npad = ncore * _NP * steps
    fields = jnp.pad(fields, ((0, 1), (0, npad - n)))

    if ncore == 1:
        return _run_shard(scan_point, fields)

    mesh = jax.sharding.Mesh(devs[:ncore], ("x",))
    P = jax.sharding.PartitionSpec

    def _shard_fn(scan_l, fields_l):
        return jax.lax.psum(_run_shard(scan_l, fields_l), "x")

    return jax.shard_map(
        _shard_fn, mesh=mesh,
        in_specs=(P(), P(None, "x")),
        out_specs=P(), check_vma=False,
    )(scan_point, fields)
